# baseline (speedup 1.0000x reference)
import functools

import jax
import jax.numpy as jnp
from jax import lax
from jax.experimental import pallas as pl
from jax.experimental.pallas import tpu as pltpu
from jax.experimental.pallas import tpu_sc as plsc

_BATCH = 16384
_DIM = 64
_CHUNK = 128


def _make_kernel(num_cores, num_subcores):
    nw = num_cores * num_subcores
    b_per_w = _BATCH // nw
    n_chunks = b_per_w // _CHUNK
    mesh = plsc.VectorSubcoreMesh(core_axis_name="c", subcore_axis_name="s")

    @functools.partial(
        pl.kernel,
        out_type=(
            jax.ShapeDtypeStruct((_BATCH, _DIM), jnp.float32),
            jax.ShapeDtypeStruct((_BATCH, _DIM), jnp.float32),
        ),
        mesh=mesh,
        compiler_params=pltpu.CompilerParams(use_tc_tiling_on_sc=False),
        scratch_types=[
            pltpu.VMEM((2 * n_chunks, _CHUNK), jnp.int32),
            pltpu.VMEM((b_per_w, _DIM), jnp.float32),
            pltpu.VMEM((b_per_w, _DIM), jnp.float32),
            pltpu.SemaphoreType.DMA,
        ],
    )
    def k(uidx_hbm, iidx_hbm, user_table, item_table, uout, iout,
          idx_v, urows_v, irows_v, sem):
        wid = lax.axis_index("s") * num_cores + lax.axis_index("c")
        row0 = wid * n_chunks
        pltpu.sync_copy(uidx_hbm.at[pl.ds(row0, n_chunks)],
                        idx_v.at[pl.ds(0, n_chunks)])
        pltpu.sync_copy(iidx_hbm.at[pl.ds(row0, n_chunks)],
                        idx_v.at[pl.ds(n_chunks, n_chunks)])
        copies = []
        for j in range(n_chunks):
            copies.append(pltpu.async_copy(
                user_table.at[idx_v.at[j]],
                urows_v.at[pl.ds(j * _CHUNK, _CHUNK)], sem))
        for j in range(n_chunks):
            copies.append(pltpu.async_copy(
                item_table.at[idx_v.at[n_chunks + j]],
                irows_v.at[pl.ds(j * _CHUNK, _CHUNK)], sem))
        for c in copies:
            c.wait()
        base = wid * b_per_w
        pltpu.sync_copy(urows_v, uout.at[pl.ds(base, b_per_w)])
        pltpu.sync_copy(irows_v, iout.at[pl.ds(base, b_per_w)])

    return k


def kernel(user_indices, item_indices, user_table, item_table):
    info = plsc.get_sparse_core_info()
    k = _make_kernel(info.num_cores, info.num_subcores)
    uidx2 = user_indices.astype(jnp.int32).reshape(-1, _CHUNK)
    iidx2 = item_indices.astype(jnp.int32).reshape(-1, _CHUNK)
    return k(uidx2, iidx2, user_table, item_table)


# speedup vs baseline: 1.4410x; 1.4410x over previous
import functools

import jax
import jax.numpy as jnp
from jax import lax
from jax.experimental import pallas as pl
from jax.experimental.pallas import tpu as pltpu
from jax.experimental.pallas import tpu_sc as plsc

_BATCH = 16384
_DIM = 64
_GRP = 16


def _make_kernel(num_cores, num_subcores):
    nw = num_cores * num_subcores
    b_per_w = _BATCH // nw
    n2 = 2 * b_per_w
    n_grp = b_per_w // _GRP
    mesh = plsc.VectorSubcoreMesh(core_axis_name="c", subcore_axis_name="s")

    @functools.partial(
        pl.kernel,
        out_type=(
            jax.ShapeDtypeStruct((_BATCH // 2, 2 * _DIM), jnp.float32),
            jax.ShapeDtypeStruct((_BATCH // 2, 2 * _DIM), jnp.float32),
        ),
        mesh=mesh,
        scratch_types=[
            pltpu.VMEM((n2,), jnp.int32),
            pltpu.VMEM((2 * _GRP, 8, _DIM), jnp.float32),
            pltpu.VMEM((b_per_w // 2, 2 * _DIM), jnp.float32),
            pltpu.SemaphoreType.DMA,
        ],
    )
    def k(uidx_hbm, iidx_hbm, utab, itab, uout, iout,
          idx_v, slab_v, out_v, sem):
        wid = lax.axis_index("s") * num_cores + lax.axis_index("c")
        base = pl.multiple_of(wid * b_per_w, b_per_w)
        pltpu.sync_copy(uidx_hbm.at[pl.ds(base, b_per_w)],
                        idx_v.at[pl.ds(0, b_per_w)])
        pltpu.sync_copy(iidx_hbm.at[pl.ds(base, b_per_w)],
                        idx_v.at[pl.ds(b_per_w, b_per_w)])
        utab3 = utab.reshape(utab.shape[0] // 8, 8, _DIM)
        itab3 = itab.reshape(itab.shape[0] // 8, 8, _DIM)

        def issue_group(tab3, jbase, ring):
            v = idx_v[pl.ds(jbase, _GRP)]
            for u in range(_GRP):
                t = v[u] >> 3
                pltpu.async_copy(tab3.at[t], slab_v.at[ring + u], sem)

        def drain_select(tab3, jbase, ring):
            for u in range(_GRP):
                pltpu.make_async_copy(
                    tab3.at[0], slab_v.at[ring + u], sem).wait()
            v = idx_v[pl.ds(jbase, _GRP)]
            for u in range(_GRP):
                rr = v[u] & 7
                half = (u & 1) * _DIM
                orow = (jbase % b_per_w) + u
                for d in range(_DIM // 16):
                    out_v[orow >> 1, pl.ds(half + d * 16, 16)] = (
                        slab_v[ring + u, rr, pl.ds(d * 16, 16)])

        for half_id in range(2):
            tab3 = utab3 if half_id == 0 else itab3
            out = uout if half_id == 0 else iout
            jb0 = half_id * b_per_w
            issue_group(tab3, jb0, 0)

            def body(g, _):
                ring = (g % 2) * _GRP
                nring = ((g + 1) % 2) * _GRP

                @pl.when(g + 1 < n_grp)
                def _issue():
                    issue_group(tab3, jb0 + (g + 1) * _GRP, nring)

                drain_select(tab3, jb0 + g * _GRP, ring)
                return _

            lax.fori_loop(0, n_grp, body, 0)
            obase = pl.multiple_of((wid * b_per_w) // 2, b_per_w // 2)
            pltpu.sync_copy(out_v, out.at[pl.ds(obase, b_per_w // 2)])

    return k


def kernel(user_indices, item_indices, user_table, item_table):
    info = plsc.get_sparse_core_info()
    k = _make_kernel(info.num_cores, info.num_subcores)
    uidx = user_indices.astype(jnp.int32)
    iidx = item_indices.astype(jnp.int32)
    u2, i2 = k(uidx, iidx, user_table, item_table)
    return (u2.reshape(_BATCH, _DIM), i2.reshape(_BATCH, _DIM))


# speedup vs baseline: 1.4675x; 1.0184x over previous
import functools

import jax
import jax.numpy as jnp
from jax import lax
from jax.experimental import pallas as pl
from jax.experimental.pallas import tpu as pltpu
from jax.experimental.pallas import tpu_sc as plsc

_BATCH = 16384
_DIM = 64
_GRP = 16


def _make_kernel(num_cores, num_subcores):
    nw = num_cores * num_subcores
    b_per_w = _BATCH // nw
    n2 = 2 * b_per_w
    n_grp = b_per_w // _GRP
    mesh = plsc.VectorSubcoreMesh(core_axis_name="c", subcore_axis_name="s")

    @functools.partial(
        pl.kernel,
        out_type=(
            jax.ShapeDtypeStruct((_BATCH, _DIM), jnp.float32),
            jax.ShapeDtypeStruct((_BATCH, _DIM), jnp.float32),
        ),
        mesh=mesh,
        scratch_types=[
            pltpu.VMEM((n2,), jnp.int32),
            pltpu.VMEM((2 * _GRP, 8, _DIM), jnp.float32),
            pltpu.VMEM((b_per_w // 8, 8, _DIM), jnp.float32),
            pltpu.SemaphoreType.DMA,
        ],
    )
    def k(uidx_hbm, iidx_hbm, utab, itab, uout, iout,
          idx_v, slab_v, out_v, sem):
        wid = lax.axis_index("s") * num_cores + lax.axis_index("c")
        base = pl.multiple_of(wid * b_per_w, b_per_w)
        pltpu.sync_copy(uidx_hbm.at[pl.ds(base, b_per_w)],
                        idx_v.at[pl.ds(0, b_per_w)])
        pltpu.sync_copy(iidx_hbm.at[pl.ds(base, b_per_w)],
                        idx_v.at[pl.ds(b_per_w, b_per_w)])
        utab3 = utab.reshape(utab.shape[0] // 8, 8, _DIM)
        itab3 = itab.reshape(itab.shape[0] // 8, 8, _DIM)

        def issue_group(tab3, jbase, ring):
            v = idx_v[pl.ds(jbase, _GRP)]
            for u in range(_GRP):
                t = v[u] >> 3
                pltpu.async_copy(tab3.at[t], slab_v.at[ring + u], sem)

        def drain_select(tab3, jbase, ring):
            pltpu.make_async_copy(
                tab3.at[pl.ds(0, _GRP)],
                slab_v.at[pl.ds(ring, _GRP)], sem).wait()
            v = idx_v[pl.ds(jbase, _GRP)]
            for u in range(_GRP):
                rr = v[u] & 7
                orow = (jbase % b_per_w) + u
                for d in range(_DIM // 16):
                    out_v[orow >> 3, u & 7, pl.ds(d * 16, 16)] = (
                        slab_v[ring + u, rr, pl.ds(d * 16, 16)])

        for half_id in range(2):
            tab3 = utab3 if half_id == 0 else itab3
            out = uout if half_id == 0 else iout
            jb0 = half_id * b_per_w
            issue_group(tab3, jb0, 0)

            def body(g, _):
                ring = (g % 2) * _GRP
                nring = ((g + 1) % 2) * _GRP

                @pl.when(g + 1 < n_grp)
                def _issue():
                    issue_group(tab3, jb0 + (g + 1) * _GRP, nring)

                drain_select(tab3, jb0 + g * _GRP, ring)
                return _

            lax.fori_loop(0, n_grp, body, 0)
            out3 = out.reshape(_BATCH // 8, 8, _DIM)
            obase = pl.multiple_of((wid * b_per_w) // 8, b_per_w // 8)
            pltpu.sync_copy(out_v, out3.at[pl.ds(obase, b_per_w // 8)])

    return k


def kernel(user_indices, item_indices, user_table, item_table):
    info = plsc.get_sparse_core_info()
    k = _make_kernel(info.num_cores, info.num_subcores)
    uidx = user_indices.astype(jnp.int32)
    iidx = item_indices.astype(jnp.int32)
    return k(uidx, iidx, user_table, item_table)


# speedup vs baseline: 1.4962x; 1.0196x over previous
import functools

import jax
import jax.numpy as jnp
from jax import lax
from jax.experimental import pallas as pl
from jax.experimental.pallas import tpu as pltpu
from jax.experimental.pallas import tpu_sc as plsc

_BATCH = 16384
_DIM = 64
_GRP = 32


def _make_kernel(num_cores, num_subcores):
    nw = num_cores * num_subcores
    b_per_w = _BATCH // nw
    n2 = 2 * b_per_w
    n_grp = b_per_w // _GRP
    gt = _GRP // 8
    mesh = plsc.VectorSubcoreMesh(core_axis_name="c", subcore_axis_name="s")

    @functools.partial(
        pl.kernel,
        out_type=(
            jax.ShapeDtypeStruct((_BATCH, _DIM), jnp.float32),
            jax.ShapeDtypeStruct((_BATCH, _DIM), jnp.float32),
        ),
        mesh=mesh,
        scratch_types=[
            pltpu.VMEM((n2,), jnp.int32),
            pltpu.VMEM((2 * _GRP, 8, _DIM), jnp.float32),
            pltpu.VMEM((2, gt, 8, _DIM), jnp.float32),
            pltpu.SemaphoreType.DMA,
            pltpu.SemaphoreType.DMA,
        ],
    )
    def k(uidx_hbm, iidx_hbm, utab, itab, uout, iout,
          idx_v, slab_v, oring_v, sem, osem):
        wid = lax.axis_index("s") * num_cores + lax.axis_index("c")
        base = pl.multiple_of(wid * b_per_w, b_per_w)
        pltpu.sync_copy(uidx_hbm.at[pl.ds(base, b_per_w)],
                        idx_v.at[pl.ds(0, b_per_w)])
        pltpu.sync_copy(iidx_hbm.at[pl.ds(base, b_per_w)],
                        idx_v.at[pl.ds(b_per_w, b_per_w)])
        utab3 = utab.reshape(utab.shape[0] // 8, 8, _DIM)
        itab3 = itab.reshape(itab.shape[0] // 8, 8, _DIM)
        uout3 = uout.reshape(_BATCH // 8, 8, _DIM)
        iout3 = iout.reshape(_BATCH // 8, 8, _DIM)

        def issue_group(tab3, jbase, ring):
            for b in range(_GRP // 16):
                v = idx_v[pl.ds(jbase + b * 16, 16)]
                for u in range(16):
                    t = v[u] >> 3
                    pltpu.async_copy(
                        tab3.at[t], slab_v.at[ring + b * 16 + u], sem)

        def select_group(jbase, ring, oslot):
            for b in range(_GRP // 16):
                v = idx_v[pl.ds(jbase + b * 16, 16)]
                for u in range(16):
                    uu = b * 16 + u
                    rr = v[u] & 7
                    for d in range(_DIM // 16):
                        oring_v[oslot, uu >> 3, uu & 7,
                                pl.ds(d * 16, 16)] = (
                            slab_v[ring + uu, rr, pl.ds(d * 16, 16)])

        for half_id in range(2):
            tab3 = utab3 if half_id == 0 else itab3
            out3 = uout3 if half_id == 0 else iout3
            jb0 = half_id * b_per_w
            otile0 = wid * (b_per_w // 8)
            issue_group(tab3, jb0, 0)

            def body(g, _):
                ring = (g % 2) * _GRP
                oslot = g % 2

                @pl.when(g + 1 < n_grp)
                def _issue():
                    issue_group(tab3, jb0 + (g + 1) * _GRP,
                                ((g + 1) % 2) * _GRP)

                pltpu.make_async_copy(
                    tab3.at[pl.ds(0, _GRP)],
                    slab_v.at[pl.ds(ring, _GRP)], sem).wait()

                @pl.when(g >= 2)
                def _owait():
                    pltpu.make_async_copy(
                        oring_v.at[oslot], out3.at[pl.ds(0, gt)],
                        osem).wait()

                select_group(jb0 + g * _GRP, ring, oslot)
                pltpu.async_copy(oring_v.at[oslot],
                                 out3.at[pl.ds(otile0 + g * gt, gt)],
                                 osem)
                return _

            lax.fori_loop(0, n_grp, body, 0)
            for tail in range(2):
                pltpu.make_async_copy(
                    oring_v.at[tail], out3.at[pl.ds(0, gt)], osem).wait()

    return k


def kernel(user_indices, item_indices, user_table, item_table):
    info = plsc.get_sparse_core_info()
    k = _make_kernel(info.num_cores, info.num_subcores)
    uidx = user_indices.astype(jnp.int32)
    iidx = item_indices.astype(jnp.int32)
    return k(uidx, iidx, user_table, item_table)
